# Initial kernel scaffold; baseline (speedup 1.0000x reference)
import jax
import jax.numpy as jnp
from jax.experimental import pallas as pl


def kernel(label_pre, bbox_pre, proposals):
    raise NotImplementedError("write your pallas kernel here")


# speedup vs baseline: 3.4145x; 3.4145x over previous
import functools

import jax
import jax.numpy as jnp
import numpy as np
from jax import lax
from jax.experimental import pallas as pl

N = 20000
C = 80
SCORE_THRESH = 0.01
NMS_THRESH = 0.5
DET_PER_IM = 100
PRE_NMS_TOPK = 1000
_BBOX_CLIP = float(np.log(1000.0 / 16.0))

_BLK = 2000


def _scores_kernel(label_ref, dx_ref, dy_ref, dw_ref, dh_ref, prop_ref, out_ref):
    lab = label_ref[...]
    m = jnp.max(lab, axis=1, keepdims=True)
    e = jnp.exp(lab - m)
    s = jnp.sum(e, axis=1, keepdims=True)
    scores = (e / s)[:, 1:]

    dx = dx_ref[...]
    dy = dy_ref[...]
    dw = jnp.minimum(dw_ref[...], _BBOX_CLIP)
    dh = jnp.minimum(dh_ref[...], _BBOX_CLIP)

    p = prop_ref[...]
    w = p[:, 2:3] - p[:, 0:1]
    h = p[:, 3:4] - p[:, 1:2]
    cx = p[:, 0:1] + 0.5 * w
    cy = p[:, 1:2] + 0.5 * h

    pcx = dx * w + cx
    pcy = dy * h + cy
    pw = jnp.exp(dw) * w
    ph = jnp.exp(dh) * h
    x1 = pcx - 0.5 * pw
    y1 = pcy - 0.5 * ph
    x2 = pcx + 0.5 * pw
    y2 = pcy + 0.5 * ph
    area = (y2 - y1) * (x2 - x1)

    valid = (scores > SCORE_THRESH) & (area > 0.1)
    out_ref[...] = jnp.where(valid, scores, -1.0)


def _masked_scores(label_pre, bbox_pre, proposals):
    grid = N // _BLK
    bbr = bbox_pre.reshape(N, C + 1, 4)
    dxs, dys, dws, dhs = (bbr[:, 1:, k] for k in range(4))
    dspec = pl.BlockSpec((_BLK, C), lambda i: (i, 0))
    return pl.pallas_call(
        _scores_kernel,
        grid=(grid,),
        in_specs=[
            pl.BlockSpec((_BLK, C + 1), lambda i: (i, 0)),
            dspec, dspec, dspec, dspec,
            pl.BlockSpec((_BLK, 4), lambda i: (i, 0)),
        ],
        out_specs=pl.BlockSpec((_BLK, C), lambda i: (i, 0)),
        out_shape=jax.ShapeDtypeStruct((N, C), jnp.float32),
    )(label_pre, dxs, dys, dws, dhs, proposals)


_NMS_P = 1024


def _nms_kernel(sc_ref, dx_ref, dy_ref, dw_ref, dh_ref,
                px1_ref, py1_ref, px2_ref, py2_ref, lbl_ref, out_ref):
    shape = (8, 128)
    rowid = lax.broadcasted_iota(jnp.int32, shape, 0)
    colid = lax.broadcasted_iota(jnp.int32, shape, 1)
    pos = rowid * 128 + colid
    slot_ok = pos < PRE_NMS_TOPK

    scores = sc_ref[...]
    w = px2_ref[...] - px1_ref[...]
    h = py2_ref[...] - py1_ref[...]
    cx = px1_ref[...] + 0.5 * w
    cy = py1_ref[...] + 0.5 * h
    dw = jnp.minimum(dw_ref[...], _BBOX_CLIP)
    dh = jnp.minimum(dh_ref[...], _BBOX_CLIP)
    pcx = dx_ref[...] * w + cx
    pcy = dy_ref[...] * h + cy
    pw = jnp.exp(dw) * w
    ph = jnp.exp(dh) * h
    x1 = pcx - 0.5 * pw
    y1 = pcy - 0.5 * ph
    x2 = pcx + 0.5 * pw
    y2 = pcy + 0.5 * ph

    neg_inf = jnp.float32(-jnp.inf)
    mc = jnp.maximum(
        jnp.max(jnp.where(slot_ok, x1, neg_inf)),
        jnp.maximum(jnp.max(jnp.where(slot_ok, y1, neg_inf)),
                    jnp.maximum(jnp.max(jnp.where(slot_ok, x2, neg_inf)),
                                jnp.max(jnp.where(slot_ok, y2, neg_inf)))))
    max_coord = mc + 1.0
    offs = lbl_ref[...] * max_coord
    x1o = x1 + offs
    y1o = y1 + offs
    x2o = x2 + offs
    y2o = y2 + offs
    areas = (x2o - x1o) * (y2o - y1o)

    work0 = jnp.where(slot_ok, scores, neg_inf)
    out0 = jnp.zeros((128, 8), jnp.float32)
    orow = lax.broadcasted_iota(jnp.int32, (128, 8), 0)
    ocol = lax.broadcasted_iota(jnp.int32, (128, 8), 1)

    def body(j, carry):
        work, out_acc = carry
        m = jnp.max(work)
        ipos = jnp.min(jnp.where(work == m, pos, jnp.int32(2**30)))
        sel = pos == ipos
        xi1 = jnp.sum(jnp.where(sel, x1o, 0.0))
        yi1 = jnp.sum(jnp.where(sel, y1o, 0.0))
        xi2 = jnp.sum(jnp.where(sel, x2o, 0.0))
        yi2 = jnp.sum(jnp.where(sel, y2o, 0.0))
        ai = jnp.sum(jnp.where(sel, areas, 0.0))
        bx1 = jnp.sum(jnp.where(sel, x1, 0.0))
        by1 = jnp.sum(jnp.where(sel, y1, 0.0))
        bx2 = jnp.sum(jnp.where(sel, x2, 0.0))
        by2 = jnp.sum(jnp.where(sel, y2, 0.0))
        valid = m > 0.0
        row = (jnp.where(ocol == 0, jnp.where(valid, bx1, 0.0), 0.0)
               + jnp.where(ocol == 1, jnp.where(valid, by1, 0.0), 0.0)
               + jnp.where(ocol == 2, jnp.where(valid, bx2, 0.0), 0.0)
               + jnp.where(ocol == 3, jnp.where(valid, by2, 0.0), 0.0)
               + jnp.where(ocol == 4, jnp.where(valid, m, 0.0), 0.0))
        out_acc = jnp.where(orow == j, row, out_acc)
        xx1 = jnp.maximum(x1o, xi1)
        yy1 = jnp.maximum(y1o, yi1)
        xx2 = jnp.minimum(x2o, xi2)
        yy2 = jnp.minimum(y2o, yi2)
        inter = jnp.clip(xx2 - xx1, 0.0) * jnp.clip(yy2 - yy1, 0.0)
        iou = inter / (areas + ai - inter + 1e-9)
        work = jnp.where(iou > NMS_THRESH, neg_inf, work)
        return work, out_acc

    _, out_acc = lax.fori_loop(0, DET_PER_IM, body, (work0, out0))
    out_ref[...] = out_acc[:DET_PER_IM, :5]


def _nms(scores, dx, dy, dw, dh, px1, py1, px2, py2, labels):
    args = [a.reshape(8, 128) for a in
            (scores, dx, dy, dw, dh, px1, py1, px2, py2, labels)]
    return pl.pallas_call(
        _nms_kernel,
        out_shape=jax.ShapeDtypeStruct((DET_PER_IM, 5), jnp.float32),
    )(*args)


@functools.partial(jax.jit, static_argnums=())
def kernel(label_pre, bbox_pre, proposals):
    masked = _masked_scores(label_pre, bbox_pre, proposals)
    top_scores, top_idx = lax.top_k(masked.reshape(-1), PRE_NMS_TOPK)
    n = top_idx // C
    c = top_idx % C
    deltas = bbox_pre.reshape(N, C + 1, 4)[n, c + 1, :]
    props = proposals[n]
    labels = (c + 1).astype(jnp.float32)

    pad = _NMS_P - PRE_NMS_TOPK
    sc_p = jnp.concatenate([top_scores, jnp.full((pad,), -1.0, jnp.float32)])
    d_p = jnp.concatenate([deltas, jnp.zeros((pad, 4), jnp.float32)])
    p_p = jnp.concatenate([props, jnp.zeros((pad, 4), jnp.float32)])
    l_p = jnp.concatenate([labels, jnp.zeros((pad,), jnp.float32)])
    return _nms(sc_p, d_p[:, 0], d_p[:, 1], d_p[:, 2], d_p[:, 3],
                p_p[:, 0], p_p[:, 1], p_p[:, 2], p_p[:, 3], l_p)


# speedup vs baseline: 19.2905x; 5.6496x over previous
import functools

import jax
import jax.numpy as jnp
import numpy as np
from jax import lax
from jax.experimental import pallas as pl
from jax.experimental.pallas import tpu as pltpu
from jax.experimental.pallas import tpu_sc as plsc

N = 20000
C = 80
SCORE_THRESH = 0.01
NMS_THRESH = 0.5
DET_PER_IM = 100
PRE_NMS_TOPK = 1000
_BBOX_CLIP = float(np.log(1000.0 / 16.0))

_BLK = 2000
_NW = 32
_NC = 2
_TOT = N * C
_CHUNK = _TOT // _NW
_NB = 2048
_HSCALE = (_NB - 2) / 0.99
_CAPC = 1024
_CAP2 = _NC * _CAPC


def _scores_kernel(label_ref, dx_ref, dy_ref, dw_ref, dh_ref, prop_ref, out_ref):
    lab = label_ref[...]
    m = jnp.max(lab, axis=1, keepdims=True)
    e = jnp.exp(lab - m)
    s = jnp.sum(e, axis=1, keepdims=True)
    scores = (e / s)[:, 1:]

    dx = dx_ref[...]
    dy = dy_ref[...]
    dw = jnp.minimum(dw_ref[...], _BBOX_CLIP)
    dh = jnp.minimum(dh_ref[...], _BBOX_CLIP)

    p = prop_ref[...]
    w = p[:, 2:3] - p[:, 0:1]
    h = p[:, 3:4] - p[:, 1:2]
    cx = p[:, 0:1] + 0.5 * w
    cy = p[:, 1:2] + 0.5 * h

    pcx = dx * w + cx
    pcy = dy * h + cy
    pw = jnp.exp(dw) * w
    ph = jnp.exp(dh) * h
    x1 = pcx - 0.5 * pw
    y1 = pcy - 0.5 * ph
    x2 = pcx + 0.5 * pw
    y2 = pcy + 0.5 * ph
    area = (y2 - y1) * (x2 - x1)

    valid = (scores > SCORE_THRESH) & (area > 0.1)
    out_ref[...] = jnp.where(valid, scores, -1.0)


def _masked_scores(label_pre, bbox_pre, proposals):
    grid = N // _BLK
    bbr = bbox_pre.reshape(N, C + 1, 4)
    dxs, dys, dws, dhs = (bbr[:, 1:, k] for k in range(4))
    dspec = pl.BlockSpec((_BLK, C), lambda i: (i, 0))
    return pl.pallas_call(
        _scores_kernel,
        grid=(grid,),
        in_specs=[
            pl.BlockSpec((_BLK, C + 1), lambda i: (i, 0)),
            dspec, dspec, dspec, dspec,
            pl.BlockSpec((_BLK, 4), lambda i: (i, 0)),
        ],
        out_specs=pl.BlockSpec((_BLK, C), lambda i: (i, 0)),
        out_shape=jax.ShapeDtypeStruct((N, C), jnp.float32),
    )(label_pre, dxs, dys, dws, dhs, proposals)


def _bucket_of(v):
    b = ((v - SCORE_THRESH) * _HSCALE).astype(jnp.int32) + 1
    return jnp.clip(b, 0, _NB - 1)


def _sc_mesh():
    return plsc.VectorSubcoreMesh(core_axis_name="c", subcore_axis_name="s",
                                  num_cores=_NC, num_subcores=_NW // _NC)


def _hist_body(sc_hbm, out_hbm, chunk_v, hist_v):
    cid = lax.axis_index("c")
    sid = lax.axis_index("s")
    wid = sid * _NC + cid
    pltpu.sync_copy(sc_hbm.at[pl.ds(pl.multiple_of(wid * _CHUNK, 8), _CHUNK)], chunk_v)

    zero = jnp.zeros((16,), jnp.int32)

    def zbody(i, carry):
        hist_v[pl.ds(i * 16, 16)] = zero
        return carry

    lax.fori_loop(0, _NB, zbody, 0)

    lanes = lax.iota(jnp.int32, 16)
    ones = jnp.ones((16,), jnp.int32)

    def body(i, carry):
        v = chunk_v[pl.ds(i * 16, 16)]
        b = _bucket_of(v)
        plsc.addupdate_scatter(hist_v, [b * 16 + lanes], ones)
        return carry

    lax.fori_loop(0, _CHUNK // 16, body, 0)
    pltpu.sync_copy(hist_v, out_hbm.at[wid])


@functools.cache
def _hist_sc():
    return pl.kernel(
        _hist_body,
        out_type=jax.ShapeDtypeStruct((_NW, _NB * 16), jnp.int32),
        mesh=_sc_mesh(),
        compiler_params=pltpu.CompilerParams(needs_layout_passes=False),
        scratch_types=[
            pltpu.VMEM((_CHUNK,), jnp.float32),
            pltpu.VMEM((_NB * 16,), jnp.int32),
        ],
    )


def _hist_call(flat):
    return _hist_sc()(flat)


def _compact_body(sc_hbm, kst_hbm, bb128_hbm, pp128_hbm, zflat_hbm,
                osc_hbm, oix_hbm, od_hbm, op_hbm,
                chunk_v, sbuf, ibuf, kst_v, zflat_v, rows_d, rows_p,
                grow_d, grow_p, cnt_smem, sem):
    cid = lax.axis_index("c")
    sid = lax.axis_index("s")
    wid = sid * _NC + cid

    neg1 = jnp.full((16,), -1.0, jnp.float32)
    izero = jnp.zeros((16,), jnp.int32)

    for t in range(4):
        sbuf[pl.ds(t * 16, 16)] = neg1
        ibuf[pl.ds(t * 16, 16)] = izero
    zoff = pl.multiple_of(cid * _CAPC + sid * (_CAPC // 16), 8)
    pltpu.sync_copy(sbuf.at[pl.ds(0, _CAPC // 16)], osc_hbm.at[pl.ds(zoff, _CAPC // 16)])
    pltpu.sync_copy(ibuf.at[pl.ds(0, _CAPC // 16)], oix_hbm.at[pl.ds(zoff, _CAPC // 16)])
    pltpu.sync_copy(zflat_hbm, zflat_v)
    for t in range(4):
        zf = pl.multiple_of(zoff * 4 + t * 64, 8)
        pltpu.sync_copy(zflat_v, od_hbm.at[pl.ds(zf, 64)])
        pltpu.sync_copy(zflat_v, op_hbm.at[pl.ds(zf, 64)])

    @pl.when(sid == 0)
    def _():
        cnt_smem[0] = 0

    pltpu.sync_copy(kst_hbm, kst_v)
    base_elem = wid * _CHUNK
    pltpu.sync_copy(sc_hbm.at[pl.ds(pl.multiple_of(base_elem, 8), _CHUNK)], chunk_v)
    plsc.subcore_barrier()

    kvec = kst_v[...]
    lanes = lax.iota(jnp.int32, 16)

    def body(i, wcnt):
        v = chunk_v[pl.ds(i * 16, 16)]
        m = _bucket_of(v) >= kvec
        pc = plsc.cumsum(jnp.where(m, 1, 0))
        cnt = jnp.max(pc)

        @pl.when(wcnt <= _CAPC - 16)
        def _():
            pos = wcnt + pc - 1
            plsc.store_scatter(sbuf, [pos], v, mask=m)
            plsc.store_scatter(ibuf, [pos], base_elem + i * 16 + lanes, mask=m)

        return jnp.minimum(wcnt + cnt, _CAPC)

    wcnt = lax.fori_loop(0, _CHUNK // 16, body, jnp.int32(0))

    plsc.store_scatter(sbuf, [wcnt + lanes], neg1)
    plsc.store_scatter(ibuf, [wcnt + lanes], izero)
    wpad = ((wcnt + 15) // 16) * 16
    mybase = plsc.fetch_and_add(cnt_smem.at[0], wpad, subcore_id=0)

    lane4 = lax.iota(jnp.int32, 16)

    def wbody(j, carry):
        off = mybase + j * 16

        @pl.when(off <= _CAPC - 16)
        def _():
            dst = pl.multiple_of(cid * _CAPC + off, 8)
            pltpu.sync_copy(sbuf.at[pl.ds(j * 16, 16)], osc_hbm.at[pl.ds(dst, 16)])
            pltpu.sync_copy(ibuf.at[pl.ds(j * 16, 16)], oix_hbm.at[pl.ds(dst, 16)])
            ivec = ibuf[pl.ds(j * 16, 16)]
            n = ivec // C
            cls = ivec - n * C
            offd = n * ((C + 1) * 4) + (cls + 1) * 4
            offp = n * 4
            pltpu.async_copy(bb128_hbm.at[lax.shift_right_logical(offd, 7)],
                             grow_d, sem).wait()
            pltpu.async_copy(pp128_hbm.at[lax.shift_right_logical(offp, 7)],
                             grow_p, sem).wait()
            cold = offd & 127
            colp = offp & 127
            for k in range(4):
                vd = plsc.load_gather(grow_d, [lane4, cold + k])
                vp = plsc.load_gather(grow_p, [lane4, colp + k])
                plsc.store_scatter(rows_d, [lane4 * 4 + k], vd)
                plsc.store_scatter(rows_p, [lane4 * 4 + k], vp)
            pltpu.sync_copy(rows_d, od_hbm.at[pl.ds(pl.multiple_of(dst * 4, 8), 64)])
            pltpu.sync_copy(rows_p, op_hbm.at[pl.ds(pl.multiple_of(dst * 4, 8), 64)])

        return carry

    lax.fori_loop(0, wpad // 16, wbody, 0)


@functools.cache
def _compact_sc():
    return pl.kernel(
        _compact_body,
        out_type=[
            jax.ShapeDtypeStruct((_CAP2,), jnp.float32),
            jax.ShapeDtypeStruct((_CAP2,), jnp.int32),
            jax.ShapeDtypeStruct((_CAP2 * 4,), jnp.float32),
            jax.ShapeDtypeStruct((_CAP2 * 4,), jnp.float32),
        ],
        mesh=_sc_mesh(),
        compiler_params=pltpu.CompilerParams(needs_layout_passes=False),
        scratch_types=[
            pltpu.VMEM((_CHUNK,), jnp.float32),
            pltpu.VMEM((_CAPC + 16,), jnp.float32),
            pltpu.VMEM((_CAPC + 16,), jnp.int32),
            pltpu.VMEM((16,), jnp.int32),
            pltpu.VMEM((64,), jnp.float32),
            pltpu.VMEM((64,), jnp.float32),
            pltpu.VMEM((64,), jnp.float32),
            pltpu.VMEM((16, 128), jnp.float32),
            pltpu.VMEM((16, 128), jnp.float32),
            pltpu.SMEM((1,), jnp.int32),
            pltpu.SemaphoreType.DMA,
        ],
    )


def _compact_call(flat, kst, bb128, pp128, zflat):
    return _compact_sc()(flat, kst, bb128, pp128, zflat)


def _nms_kernel(sc_ref, ix_ref, dx_ref, dy_ref, dw_ref, dh_ref,
                px1_ref, py1_ref, px2_ref, py2_ref, out_ref):
    shape = (_CAP2 // 128, 128)
    scores = sc_ref[...]
    ix = ix_ref[...]

    def vbody(t, lh):
        lo, hi = lh
        mid = 0.5 * (lo + hi)
        cnt = jnp.sum(jnp.where(scores >= mid, 1, 0))
        big = cnt >= PRE_NMS_TOPK
        return jnp.where(big, mid, lo), jnp.where(big, hi, mid)

    vstar, _ = lax.fori_loop(0, 64, vbody, (jnp.float32(-2.0), jnp.float32(2.0)))
    gt = scores > vstar
    ties = scores == vstar
    need = PRE_NMS_TOPK - jnp.sum(jnp.where(gt, 1, 0))

    def tbody(t, T):
        Tp = T + lax.shift_left(jnp.int32(1), 20 - t)
        cnt = jnp.sum(jnp.where(ties & (ix < Tp), 1, 0))
        return jnp.where(cnt <= need, Tp, T)

    tstar = lax.fori_loop(0, 21, tbody, jnp.int32(0))
    kept = gt | (ties & (ix < tstar))

    w = px2_ref[...] - px1_ref[...]
    h = py2_ref[...] - py1_ref[...]
    cx = px1_ref[...] + 0.5 * w
    cy = py1_ref[...] + 0.5 * h
    dw = jnp.minimum(dw_ref[...], _BBOX_CLIP)
    dh = jnp.minimum(dh_ref[...], _BBOX_CLIP)
    pcx = dx_ref[...] * w + cx
    pcy = dy_ref[...] * h + cy
    pw = jnp.exp(dw) * w
    ph = jnp.exp(dh) * h
    x1 = pcx - 0.5 * pw
    y1 = pcy - 0.5 * ph
    x2 = pcx + 0.5 * pw
    y2 = pcy + 0.5 * ph

    neg_inf = jnp.float32(-jnp.inf)
    mc = jnp.maximum(
        jnp.max(jnp.where(kept, x1, neg_inf)),
        jnp.maximum(jnp.max(jnp.where(kept, y1, neg_inf)),
                    jnp.maximum(jnp.max(jnp.where(kept, x2, neg_inf)),
                                jnp.max(jnp.where(kept, y2, neg_inf)))))
    max_coord = mc + 1.0
    labels = ((ix - (ix // C) * C) + 1).astype(jnp.float32)
    offs = labels * max_coord
    x1o = x1 + offs
    y1o = y1 + offs
    x2o = x2 + offs
    y2o = y2 + offs
    areas = (x2o - x1o) * (y2o - y1o)

    work0 = jnp.where(kept, scores, neg_inf)
    out0 = jnp.zeros((128, 8), jnp.float32)
    orow = lax.broadcasted_iota(jnp.int32, (128, 8), 0)
    ocol = lax.broadcasted_iota(jnp.int32, (128, 8), 1)
    big_i = jnp.int32(2**30)

    def body(j, carry):
        work, out_acc = carry
        m = jnp.max(work)
        iix = jnp.min(jnp.where(work == m, ix, big_i))
        sel = (work == m) & (ix == iix)
        xi1 = jnp.sum(jnp.where(sel, x1o, 0.0))
        yi1 = jnp.sum(jnp.where(sel, y1o, 0.0))
        xi2 = jnp.sum(jnp.where(sel, x2o, 0.0))
        yi2 = jnp.sum(jnp.where(sel, y2o, 0.0))
        ai = jnp.sum(jnp.where(sel, areas, 0.0))
        bx1 = jnp.sum(jnp.where(sel, x1, 0.0))
        by1 = jnp.sum(jnp.where(sel, y1, 0.0))
        bx2 = jnp.sum(jnp.where(sel, x2, 0.0))
        by2 = jnp.sum(jnp.where(sel, y2, 0.0))
        valid = m > 0.0
        row = (jnp.where(ocol == 0, jnp.where(valid, bx1, 0.0), 0.0)
               + jnp.where(ocol == 1, jnp.where(valid, by1, 0.0), 0.0)
               + jnp.where(ocol == 2, jnp.where(valid, bx2, 0.0), 0.0)
               + jnp.where(ocol == 3, jnp.where(valid, by2, 0.0), 0.0)
               + jnp.where(ocol == 4, jnp.where(valid, m, 0.0), 0.0))
        out_acc = jnp.where(orow == j, row, out_acc)
        xx1 = jnp.maximum(x1o, xi1)
        yy1 = jnp.maximum(y1o, yi1)
        xx2 = jnp.minimum(x2o, xi2)
        yy2 = jnp.minimum(y2o, yi2)
        inter = jnp.clip(xx2 - xx1, 0.0) * jnp.clip(yy2 - yy1, 0.0)
        iou = inter / (areas + ai - inter + 1e-9)
        work = jnp.where(iou > NMS_THRESH, neg_inf, work)
        return work, out_acc

    _, out_acc = lax.fori_loop(0, DET_PER_IM, body, (work0, out0))
    out_ref[...] = out_acc[:DET_PER_IM, :5]


def _nms(csc, cix, cd, cp):
    shp = (_CAP2 // 128, 128)
    args = [csc.reshape(shp), cix.reshape(shp)]
    args += [cd[:, k].reshape(shp) for k in range(4)]
    args += [cp[:, k].reshape(shp) for k in range(4)]
    return pl.pallas_call(
        _nms_kernel,
        out_shape=jax.ShapeDtypeStruct((DET_PER_IM, 5), jnp.float32),
    )(*args)


@jax.jit
def kernel(label_pre, bbox_pre, proposals):
    masked = _masked_scores(label_pre, bbox_pre, proposals)
    flat = masked.reshape(_TOT)

    hist = _hist_call(flat)
    counts = hist.reshape(_NW, _NB, 16).sum(axis=(0, 2))
    cum = jnp.cumsum(counts[::-1])[::-1]
    ks = jnp.arange(_NB, dtype=jnp.int32)
    feas = (cum >= PRE_NMS_TOPK) & (ks >= 1)
    kstar = jnp.max(jnp.where(feas, ks, 1)).astype(jnp.int32)

    csc, cix, cdf, cpf = _compact_call(
        flat,
        jnp.full((16,), kstar, jnp.int32),
        bbox_pre.reshape(N * (C + 1) * 4 // 128, 128),
        proposals.reshape(N * 4 // 128, 128),
        jnp.zeros((64,), jnp.float32),
    )
    return _nms(csc, cix, cdf.reshape(_CAP2, 4), cpf.reshape(_CAP2, 4))


# speedup vs baseline: 21.2397x; 1.1010x over previous
import functools

import jax
import jax.numpy as jnp
import numpy as np
from jax import lax
from jax.experimental import pallas as pl
from jax.experimental.pallas import tpu as pltpu
from jax.experimental.pallas import tpu_sc as plsc

N = 20000
C = 80
SCORE_THRESH = 0.01
NMS_THRESH = 0.5
DET_PER_IM = 100
PRE_NMS_TOPK = 1000
_BBOX_CLIP = float(np.log(1000.0 / 16.0))

_BLK = 2000
_NW = 32
_NC = 2
_TOT = N * C
_CHUNK = _TOT // _NW
_NB = 2048
_HSCALE = (_NB - 2) / 0.99
_CAPC = 1024
_CAP2 = _NC * _CAPC


def _scores_kernel(label_ref, bbox_ref, prop_ref, out_ref):
    lab = label_ref[...]
    m = jnp.max(lab, axis=1, keepdims=True)
    e = jnp.exp(lab - m)
    s = jnp.sum(e, axis=1, keepdims=True)
    scores = (e / s)[:, 1:]

    bb = bbox_ref[...]
    rr = lax.broadcasted_iota(jnp.int32, ((C + 1) * 4, C), 0)
    cc = lax.broadcasted_iota(jnp.int32, ((C + 1) * 4, C), 1)

    def _sel(k):
        s = (rr == (cc + 1) * 4 + k).astype(jnp.float32)
        return lax.dot_general(bb, s, (((1,), (0,)), ((), ())),
                               precision=lax.Precision.HIGHEST)

    dx = _sel(0)
    dy = _sel(1)
    dw = jnp.minimum(_sel(2), _BBOX_CLIP)
    dh = jnp.minimum(_sel(3), _BBOX_CLIP)

    p = prop_ref[...]
    w = p[:, 2:3] - p[:, 0:1]
    h = p[:, 3:4] - p[:, 1:2]
    cx = p[:, 0:1] + 0.5 * w
    cy = p[:, 1:2] + 0.5 * h

    pcx = dx * w + cx
    pcy = dy * h + cy
    pw = jnp.exp(dw) * w
    ph = jnp.exp(dh) * h
    x1 = pcx - 0.5 * pw
    y1 = pcy - 0.5 * ph
    x2 = pcx + 0.5 * pw
    y2 = pcy + 0.5 * ph
    area = (y2 - y1) * (x2 - x1)

    valid = (scores > SCORE_THRESH) & (area > 0.1)
    out_ref[...] = jnp.where(valid, scores, -1.0)


def _masked_scores(label_pre, bbox_pre, proposals):
    grid = N // _BLK
    return pl.pallas_call(
        _scores_kernel,
        grid=(grid,),
        in_specs=[
            pl.BlockSpec((_BLK, C + 1), lambda i: (i, 0)),
            pl.BlockSpec((_BLK, (C + 1) * 4), lambda i: (i, 0)),
            pl.BlockSpec((_BLK, 4), lambda i: (i, 0)),
        ],
        out_specs=pl.BlockSpec((_BLK, C), lambda i: (i, 0)),
        out_shape=jax.ShapeDtypeStruct((N, C), jnp.float32),
    )(label_pre, bbox_pre, proposals)


def _bucket_of(v):
    b = ((v - SCORE_THRESH) * _HSCALE).astype(jnp.int32) + 1
    return jnp.clip(b, 0, _NB - 1)


def _sc_mesh():
    return plsc.VectorSubcoreMesh(core_axis_name="c", subcore_axis_name="s",
                                  num_cores=_NC, num_subcores=_NW // _NC)


def _hist_body(sc_hbm, out_hbm, chunk_v, hist_v):
    cid = lax.axis_index("c")
    sid = lax.axis_index("s")
    wid = sid * _NC + cid
    pltpu.sync_copy(sc_hbm.at[pl.ds(pl.multiple_of(wid * _CHUNK, 8), _CHUNK)], chunk_v)

    zero = jnp.zeros((16,), jnp.int32)

    def zbody(i, carry):
        hist_v[pl.ds(i * 16, 16)] = zero
        return carry

    lax.fori_loop(0, _NB, zbody, 0)

    lanes = lax.iota(jnp.int32, 16)
    ones = jnp.ones((16,), jnp.int32)

    def body(i, carry):
        v = chunk_v[pl.ds(i * 16, 16)]
        b = _bucket_of(v)
        plsc.addupdate_scatter(hist_v, [b * 16 + lanes], ones)
        return carry

    lax.fori_loop(0, _CHUNK // 16, body, 0)
    pltpu.sync_copy(hist_v, out_hbm.at[wid])


@functools.cache
def _hist_sc():
    return pl.kernel(
        _hist_body,
        out_type=jax.ShapeDtypeStruct((_NW, _NB * 16), jnp.int32),
        mesh=_sc_mesh(),
        compiler_params=pltpu.CompilerParams(needs_layout_passes=False),
        scratch_types=[
            pltpu.VMEM((_CHUNK,), jnp.float32),
            pltpu.VMEM((_NB * 16,), jnp.int32),
        ],
    )


def _hist_call(flat):
    return _hist_sc()(flat)


def _compact_body(sc_hbm, kst_hbm, bb128_hbm, pp128_hbm, zflat_hbm,
                osc_hbm, oix_hbm, od_hbm, op_hbm,
                chunk_v, sbuf, ibuf, kst_v, zflat_v, rows_d, rows_p,
                grow_d, grow_p, cnt_smem, sem):
    cid = lax.axis_index("c")
    sid = lax.axis_index("s")
    wid = sid * _NC + cid

    neg1 = jnp.full((16,), -1.0, jnp.float32)
    izero = jnp.zeros((16,), jnp.int32)

    for t in range(4):
        sbuf[pl.ds(t * 16, 16)] = neg1
        ibuf[pl.ds(t * 16, 16)] = izero
    zoff = pl.multiple_of(cid * _CAPC + sid * (_CAPC // 16), 8)
    pltpu.sync_copy(sbuf.at[pl.ds(0, _CAPC // 16)], osc_hbm.at[pl.ds(zoff, _CAPC // 16)])
    pltpu.sync_copy(ibuf.at[pl.ds(0, _CAPC // 16)], oix_hbm.at[pl.ds(zoff, _CAPC // 16)])
    pltpu.sync_copy(zflat_hbm, zflat_v)
    for t in range(4):
        zf = pl.multiple_of(zoff * 4 + t * 64, 8)
        pltpu.sync_copy(zflat_v, od_hbm.at[pl.ds(zf, 64)])
        pltpu.sync_copy(zflat_v, op_hbm.at[pl.ds(zf, 64)])

    @pl.when(sid == 0)
    def _():
        cnt_smem[0] = 0

    pltpu.sync_copy(kst_hbm, kst_v)
    base_elem = wid * _CHUNK
    pltpu.sync_copy(sc_hbm.at[pl.ds(pl.multiple_of(base_elem, 8), _CHUNK)], chunk_v)
    plsc.subcore_barrier()

    kvec = kst_v[...]
    lanes = lax.iota(jnp.int32, 16)

    def body(i, wcnt):
        v = chunk_v[pl.ds(i * 16, 16)]
        m = _bucket_of(v) >= kvec
        pc = plsc.cumsum(jnp.where(m, 1, 0))
        cnt = jnp.max(pc)

        @pl.when(wcnt <= _CAPC - 16)
        def _():
            pos = wcnt + pc - 1
            plsc.store_scatter(sbuf, [pos], v, mask=m)
            plsc.store_scatter(ibuf, [pos], base_elem + i * 16 + lanes, mask=m)

        return jnp.minimum(wcnt + cnt, _CAPC)

    wcnt = lax.fori_loop(0, _CHUNK // 16, body, jnp.int32(0))

    plsc.store_scatter(sbuf, [wcnt + lanes], neg1)
    plsc.store_scatter(ibuf, [wcnt + lanes], izero)
    wpad = ((wcnt + 15) // 16) * 16
    mybase = plsc.fetch_and_add(cnt_smem.at[0], wpad, subcore_id=0)

    lane4 = lax.iota(jnp.int32, 16)

    def wbody(j, carry):
        off = mybase + j * 16

        @pl.when(off <= _CAPC - 16)
        def _():
            dst = pl.multiple_of(cid * _CAPC + off, 8)
            pltpu.sync_copy(sbuf.at[pl.ds(j * 16, 16)], osc_hbm.at[pl.ds(dst, 16)])
            pltpu.sync_copy(ibuf.at[pl.ds(j * 16, 16)], oix_hbm.at[pl.ds(dst, 16)])
            ivec = ibuf[pl.ds(j * 16, 16)]
            n = ivec // C
            cls = ivec - n * C
            offd = n * ((C + 1) * 4) + (cls + 1) * 4
            offp = n * 4
            pltpu.async_copy(bb128_hbm.at[lax.shift_right_logical(offd, 7)],
                             grow_d, sem).wait()
            pltpu.async_copy(pp128_hbm.at[lax.shift_right_logical(offp, 7)],
                             grow_p, sem).wait()
            cold = offd & 127
            colp = offp & 127
            for k in range(4):
                vd = plsc.load_gather(grow_d, [lane4, cold + k])
                vp = plsc.load_gather(grow_p, [lane4, colp + k])
                plsc.store_scatter(rows_d, [lane4 * 4 + k], vd)
                plsc.store_scatter(rows_p, [lane4 * 4 + k], vp)
            pltpu.sync_copy(rows_d, od_hbm.at[pl.ds(pl.multiple_of(dst * 4, 8), 64)])
            pltpu.sync_copy(rows_p, op_hbm.at[pl.ds(pl.multiple_of(dst * 4, 8), 64)])

        return carry

    lax.fori_loop(0, wpad // 16, wbody, 0)


@functools.cache
def _compact_sc():
    return pl.kernel(
        _compact_body,
        out_type=[
            jax.ShapeDtypeStruct((_CAP2,), jnp.float32),
            jax.ShapeDtypeStruct((_CAP2,), jnp.int32),
            jax.ShapeDtypeStruct((_CAP2 * 4,), jnp.float32),
            jax.ShapeDtypeStruct((_CAP2 * 4,), jnp.float32),
        ],
        mesh=_sc_mesh(),
        compiler_params=pltpu.CompilerParams(needs_layout_passes=False),
        scratch_types=[
            pltpu.VMEM((_CHUNK,), jnp.float32),
            pltpu.VMEM((_CAPC + 16,), jnp.float32),
            pltpu.VMEM((_CAPC + 16,), jnp.int32),
            pltpu.VMEM((16,), jnp.int32),
            pltpu.VMEM((64,), jnp.float32),
            pltpu.VMEM((64,), jnp.float32),
            pltpu.VMEM((64,), jnp.float32),
            pltpu.VMEM((16, 128), jnp.float32),
            pltpu.VMEM((16, 128), jnp.float32),
            pltpu.SMEM((1,), jnp.int32),
            pltpu.SemaphoreType.DMA,
        ],
    )


def _compact_call(flat, kst, bb128, pp128, zflat):
    return _compact_sc()(flat, kst, bb128, pp128, zflat)


def _nms_kernel(sc_ref, ix_ref, dx_ref, dy_ref, dw_ref, dh_ref,
                px1_ref, py1_ref, px2_ref, py2_ref, out_ref):
    shape = (_CAP2 // 128, 128)
    scores = sc_ref[...]
    ix = ix_ref[...]

    def vbody(t, lh):
        lo, hi = lh
        mid = 0.5 * (lo + hi)
        cnt = jnp.sum(jnp.where(scores >= mid, 1, 0))
        big = cnt >= PRE_NMS_TOPK
        return jnp.where(big, mid, lo), jnp.where(big, hi, mid)

    vstar, _ = lax.fori_loop(0, 64, vbody, (jnp.float32(-2.0), jnp.float32(2.0)))
    gt = scores > vstar
    ties = scores == vstar
    need = PRE_NMS_TOPK - jnp.sum(jnp.where(gt, 1, 0))

    def tbody(t, T):
        Tp = T + lax.shift_left(jnp.int32(1), 20 - t)
        cnt = jnp.sum(jnp.where(ties & (ix < Tp), 1, 0))
        return jnp.where(cnt <= need, Tp, T)

    tstar = lax.fori_loop(0, 21, tbody, jnp.int32(0))
    kept = gt | (ties & (ix < tstar))

    w = px2_ref[...] - px1_ref[...]
    h = py2_ref[...] - py1_ref[...]
    cx = px1_ref[...] + 0.5 * w
    cy = py1_ref[...] + 0.5 * h
    dw = jnp.minimum(dw_ref[...], _BBOX_CLIP)
    dh = jnp.minimum(dh_ref[...], _BBOX_CLIP)
    pcx = dx_ref[...] * w + cx
    pcy = dy_ref[...] * h + cy
    pw = jnp.exp(dw) * w
    ph = jnp.exp(dh) * h
    x1 = pcx - 0.5 * pw
    y1 = pcy - 0.5 * ph
    x2 = pcx + 0.5 * pw
    y2 = pcy + 0.5 * ph

    neg_inf = jnp.float32(-jnp.inf)
    mc = jnp.maximum(
        jnp.max(jnp.where(kept, x1, neg_inf)),
        jnp.maximum(jnp.max(jnp.where(kept, y1, neg_inf)),
                    jnp.maximum(jnp.max(jnp.where(kept, x2, neg_inf)),
                                jnp.max(jnp.where(kept, y2, neg_inf)))))
    max_coord = mc + 1.0
    labels = ((ix - (ix // C) * C) + 1).astype(jnp.float32)
    offs = labels * max_coord
    x1o = x1 + offs
    y1o = y1 + offs
    x2o = x2 + offs
    y2o = y2 + offs
    areas = (x2o - x1o) * (y2o - y1o)

    work0 = jnp.where(kept, scores, neg_inf)
    out0 = jnp.zeros((128, 8), jnp.float32)
    orow = lax.broadcasted_iota(jnp.int32, (128, 8), 0)
    ocol = lax.broadcasted_iota(jnp.int32, (128, 8), 1)
    big_i = jnp.int32(2**30)

    def body(j, carry):
        work, out_acc = carry
        m = jnp.max(work)
        iix = jnp.min(jnp.where(work == m, ix, big_i))
        sel = (work == m) & (ix == iix)
        xi1 = jnp.sum(jnp.where(sel, x1o, 0.0))
        yi1 = jnp.sum(jnp.where(sel, y1o, 0.0))
        xi2 = jnp.sum(jnp.where(sel, x2o, 0.0))
        yi2 = jnp.sum(jnp.where(sel, y2o, 0.0))
        ai = jnp.sum(jnp.where(sel, areas, 0.0))
        bx1 = jnp.sum(jnp.where(sel, x1, 0.0))
        by1 = jnp.sum(jnp.where(sel, y1, 0.0))
        bx2 = jnp.sum(jnp.where(sel, x2, 0.0))
        by2 = jnp.sum(jnp.where(sel, y2, 0.0))
        valid = m > 0.0
        row = (jnp.where(ocol == 0, jnp.where(valid, bx1, 0.0), 0.0)
               + jnp.where(ocol == 1, jnp.where(valid, by1, 0.0), 0.0)
               + jnp.where(ocol == 2, jnp.where(valid, bx2, 0.0), 0.0)
               + jnp.where(ocol == 3, jnp.where(valid, by2, 0.0), 0.0)
               + jnp.where(ocol == 4, jnp.where(valid, m, 0.0), 0.0))
        out_acc = jnp.where(orow == j, row, out_acc)
        xx1 = jnp.maximum(x1o, xi1)
        yy1 = jnp.maximum(y1o, yi1)
        xx2 = jnp.minimum(x2o, xi2)
        yy2 = jnp.minimum(y2o, yi2)
        inter = jnp.clip(xx2 - xx1, 0.0) * jnp.clip(yy2 - yy1, 0.0)
        iou = inter / (areas + ai - inter + 1e-9)
        work = jnp.where(iou > NMS_THRESH, neg_inf, work)
        return work, out_acc

    _, out_acc = lax.fori_loop(0, DET_PER_IM, body, (work0, out0))
    out_ref[...] = out_acc[:DET_PER_IM, :5]


def _nms(csc, cix, cd, cp):
    shp = (_CAP2 // 128, 128)
    args = [csc.reshape(shp), cix.reshape(shp)]
    args += [cd[:, k].reshape(shp) for k in range(4)]
    args += [cp[:, k].reshape(shp) for k in range(4)]
    return pl.pallas_call(
        _nms_kernel,
        out_shape=jax.ShapeDtypeStruct((DET_PER_IM, 5), jnp.float32),
    )(*args)


@jax.jit
def kernel(label_pre, bbox_pre, proposals):
    masked = _masked_scores(label_pre, bbox_pre, proposals)
    flat = masked.reshape(_TOT)

    hist = _hist_call(flat)
    counts = hist.reshape(_NW, _NB, 16).sum(axis=(0, 2))
    cum = jnp.cumsum(counts[::-1])[::-1]
    ks = jnp.arange(_NB, dtype=jnp.int32)
    feas = (cum >= PRE_NMS_TOPK) & (ks >= 1)
    kstar = jnp.max(jnp.where(feas, ks, 1)).astype(jnp.int32)

    csc, cix, cdf, cpf = _compact_call(
        flat,
        jnp.full((16,), kstar, jnp.int32),
        bbox_pre.reshape(N * (C + 1) * 4 // 128, 128),
        proposals.reshape(N * 4 // 128, 128),
        jnp.zeros((64,), jnp.float32),
    )
    return _nms(csc, cix, cdf.reshape(_CAP2, 4), cpf.reshape(_CAP2, 4))


# speedup vs baseline: 24.9548x; 1.1749x over previous
import functools

import jax
import jax.numpy as jnp
import numpy as np
from jax import lax
from jax.experimental import pallas as pl
from jax.experimental.pallas import tpu as pltpu
from jax.experimental.pallas import tpu_sc as plsc

N = 20000
C = 80
SCORE_THRESH = 0.01
NMS_THRESH = 0.5
DET_PER_IM = 100
PRE_NMS_TOPK = 1000
_BBOX_CLIP = float(np.log(1000.0 / 16.0))

_BLK = 2000
_NW = 32
_NC = 2
_TOT = N * C
_CHUNK = _TOT // _NW
_NB = 2048
_HSCALE = (_NB - 2) / 0.99
_CAPC = 1024
_CAP2 = _NC * _CAPC


def _scores_kernel(label_ref, bbox_ref, prop_ref, out_ref):
    lab = label_ref[...]
    m = jnp.max(lab, axis=1, keepdims=True)
    e = jnp.exp(lab - m)
    s = jnp.sum(e, axis=1, keepdims=True)
    scores = (e / s)[:, 1:]

    bb = bbox_ref[...]
    rr = lax.broadcasted_iota(jnp.int32, ((C + 1) * 4, C), 0)
    cc = lax.broadcasted_iota(jnp.int32, ((C + 1) * 4, C), 1)

    def _sel(k):
        s = (rr == (cc + 1) * 4 + k).astype(jnp.float32)
        return lax.dot_general(bb, s, (((1,), (0,)), ((), ())),
                               precision=lax.Precision.HIGHEST)

    dx = _sel(0)
    dy = _sel(1)
    dw = jnp.minimum(_sel(2), _BBOX_CLIP)
    dh = jnp.minimum(_sel(3), _BBOX_CLIP)

    p = prop_ref[...]
    w = p[:, 2:3] - p[:, 0:1]
    h = p[:, 3:4] - p[:, 1:2]
    cx = p[:, 0:1] + 0.5 * w
    cy = p[:, 1:2] + 0.5 * h

    pcx = dx * w + cx
    pcy = dy * h + cy
    pw = jnp.exp(dw) * w
    ph = jnp.exp(dh) * h
    x1 = pcx - 0.5 * pw
    y1 = pcy - 0.5 * ph
    x2 = pcx + 0.5 * pw
    y2 = pcy + 0.5 * ph
    area = (y2 - y1) * (x2 - x1)

    valid = (scores > SCORE_THRESH) & (area > 0.1)
    out_ref[...] = jnp.where(valid, scores, -1.0)


def _masked_scores(label_pre, bbox_pre, proposals):
    grid = N // _BLK
    return pl.pallas_call(
        _scores_kernel,
        grid=(grid,),
        in_specs=[
            pl.BlockSpec((_BLK, C + 1), lambda i: (i, 0)),
            pl.BlockSpec((_BLK, (C + 1) * 4), lambda i: (i, 0)),
            pl.BlockSpec((_BLK, 4), lambda i: (i, 0)),
        ],
        out_specs=pl.BlockSpec((_BLK, C), lambda i: (i, 0)),
        out_shape=jax.ShapeDtypeStruct((N, C), jnp.float32),
    )(label_pre, bbox_pre, proposals)


def _bucket_of(v):
    b = ((v - SCORE_THRESH) * _HSCALE).astype(jnp.int32) + 1
    return jnp.clip(b, 0, _NB - 1)


def _sc_mesh():
    return plsc.VectorSubcoreMesh(core_axis_name="c", subcore_axis_name="s",
                                  num_cores=_NC, num_subcores=_NW // _NC)


def _hist_body(sc_hbm, out_hbm, chunk_v, hist_v):
    cid = lax.axis_index("c")
    sid = lax.axis_index("s")
    wid = sid * _NC + cid
    pltpu.sync_copy(sc_hbm.at[pl.ds(pl.multiple_of(wid * _CHUNK, 8), _CHUNK)], chunk_v)

    zero = jnp.zeros((16,), jnp.int32)

    def zbody(i, carry):
        hist_v[pl.ds(i * 16, 16)] = zero
        return carry

    lax.fori_loop(0, _NB, zbody, 0)

    lanes = lax.iota(jnp.int32, 16)
    ones = jnp.ones((16,), jnp.int32)

    @plsc.parallel_loop(0, _CHUNK // 16, unroll=8)
    def _(i):
        v = chunk_v[pl.ds(i * 16, 16)]
        b = _bucket_of(v)
        plsc.addupdate_scatter(hist_v, [b * 16 + lanes], ones)
    pltpu.sync_copy(hist_v, out_hbm.at[wid])


@functools.cache
def _hist_sc():
    return pl.kernel(
        _hist_body,
        out_type=jax.ShapeDtypeStruct((_NW, _NB * 16), jnp.int32),
        mesh=_sc_mesh(),
        compiler_params=pltpu.CompilerParams(needs_layout_passes=False),
        scratch_types=[
            pltpu.VMEM((_CHUNK,), jnp.float32),
            pltpu.VMEM((_NB * 16,), jnp.int32),
        ],
    )


def _hist_call(flat):
    return _hist_sc()(flat)


def _compact_body(sc_hbm, kst_hbm, bb128_hbm, pp128_hbm, zflat_hbm,
                osc_hbm, oix_hbm, od_hbm, op_hbm,
                chunk_v, sbuf, ibuf, kst_v, zflat_v, rows_d, rows_p,
                grow_d, grow_p, cnt_smem, sem):
    cid = lax.axis_index("c")
    sid = lax.axis_index("s")
    wid = sid * _NC + cid

    neg1 = jnp.full((16,), -1.0, jnp.float32)
    izero = jnp.zeros((16,), jnp.int32)

    for t in range(4):
        sbuf[pl.ds(t * 16, 16)] = neg1
        ibuf[pl.ds(t * 16, 16)] = izero
    zoff = pl.multiple_of(cid * _CAPC + sid * (_CAPC // 16), 8)
    pltpu.sync_copy(sbuf.at[pl.ds(0, _CAPC // 16)], osc_hbm.at[pl.ds(zoff, _CAPC // 16)])
    pltpu.sync_copy(ibuf.at[pl.ds(0, _CAPC // 16)], oix_hbm.at[pl.ds(zoff, _CAPC // 16)])
    pltpu.sync_copy(zflat_hbm, zflat_v)
    for t in range(4):
        zf = pl.multiple_of(zoff * 4 + t * 64, 8)
        pltpu.sync_copy(zflat_v, od_hbm.at[pl.ds(zf, 64)])
        pltpu.sync_copy(zflat_v, op_hbm.at[pl.ds(zf, 64)])

    @pl.when(sid == 0)
    def _():
        cnt_smem[0] = 0

    pltpu.sync_copy(kst_hbm, kst_v)
    base_elem = wid * _CHUNK
    pltpu.sync_copy(sc_hbm.at[pl.ds(pl.multiple_of(base_elem, 8), _CHUNK)], chunk_v)
    plsc.subcore_barrier()

    kvec = kst_v[...]
    lanes = lax.iota(jnp.int32, 16)

    capv = jnp.full((16,), _CAPC, jnp.int32)

    @plsc.parallel_loop(0, _CHUNK // 16, unroll=4,
                        carry=jnp.zeros((16,), jnp.int32))
    def wvec(i, wv):
        v = chunk_v[pl.ds(i * 16, 16)]
        m = _bucket_of(v) >= kvec
        pc = plsc.cumsum(jnp.where(m, 1, 0))
        pos = wv + pc - 1
        plsc.store_scatter(sbuf, [pos], v, mask=m)
        plsc.store_scatter(ibuf, [pos], base_elem + i * 16 + lanes, mask=m)
        return jnp.minimum(wv + plsc.all_reduce_population_count(m), capv)

    wcnt = jnp.max(wvec)

    plsc.store_scatter(sbuf, [wcnt + lanes], neg1)
    plsc.store_scatter(ibuf, [wcnt + lanes], izero)
    wpad = ((wcnt + 15) // 16) * 16
    mybase = plsc.fetch_and_add(cnt_smem.at[0], wpad, subcore_id=0)

    lane4 = lax.iota(jnp.int32, 16)

    def wbody(j, carry):
        off = mybase + j * 16

        @pl.when(off <= _CAPC - 16)
        def _():
            dst = pl.multiple_of(cid * _CAPC + off, 8)
            pltpu.sync_copy(sbuf.at[pl.ds(j * 16, 16)], osc_hbm.at[pl.ds(dst, 16)])
            pltpu.sync_copy(ibuf.at[pl.ds(j * 16, 16)], oix_hbm.at[pl.ds(dst, 16)])
            ivec = ibuf[pl.ds(j * 16, 16)]
            n = ivec // C
            cls = ivec - n * C
            offd = n * ((C + 1) * 4) + (cls + 1) * 4
            offp = n * 4
            pltpu.async_copy(bb128_hbm.at[lax.shift_right_logical(offd, 7)],
                             grow_d, sem).wait()
            pltpu.async_copy(pp128_hbm.at[lax.shift_right_logical(offp, 7)],
                             grow_p, sem).wait()
            cold = offd & 127
            colp = offp & 127
            for k in range(4):
                vd = plsc.load_gather(grow_d, [lane4, cold + k])
                vp = plsc.load_gather(grow_p, [lane4, colp + k])
                plsc.store_scatter(rows_d, [lane4 * 4 + k], vd)
                plsc.store_scatter(rows_p, [lane4 * 4 + k], vp)
            pltpu.sync_copy(rows_d, od_hbm.at[pl.ds(pl.multiple_of(dst * 4, 8), 64)])
            pltpu.sync_copy(rows_p, op_hbm.at[pl.ds(pl.multiple_of(dst * 4, 8), 64)])

        return carry

    lax.fori_loop(0, wpad // 16, wbody, 0)


@functools.cache
def _compact_sc():
    return pl.kernel(
        _compact_body,
        out_type=[
            jax.ShapeDtypeStruct((_CAP2,), jnp.float32),
            jax.ShapeDtypeStruct((_CAP2,), jnp.int32),
            jax.ShapeDtypeStruct((_CAP2 * 4,), jnp.float32),
            jax.ShapeDtypeStruct((_CAP2 * 4,), jnp.float32),
        ],
        mesh=_sc_mesh(),
        compiler_params=pltpu.CompilerParams(needs_layout_passes=False),
        scratch_types=[
            pltpu.VMEM((_CHUNK,), jnp.float32),
            pltpu.VMEM((_CAPC + 16,), jnp.float32),
            pltpu.VMEM((_CAPC + 16,), jnp.int32),
            pltpu.VMEM((16,), jnp.int32),
            pltpu.VMEM((64,), jnp.float32),
            pltpu.VMEM((64,), jnp.float32),
            pltpu.VMEM((64,), jnp.float32),
            pltpu.VMEM((16, 128), jnp.float32),
            pltpu.VMEM((16, 128), jnp.float32),
            pltpu.SMEM((1,), jnp.int32),
            pltpu.SemaphoreType.DMA,
        ],
    )


def _compact_call(flat, kst, bb128, pp128, zflat):
    return _compact_sc()(flat, kst, bb128, pp128, zflat)


def _nms_kernel(sc_ref, ix_ref, dx_ref, dy_ref, dw_ref, dh_ref,
                px1_ref, py1_ref, px2_ref, py2_ref, out_ref):
    shape = (_CAP2 // 128, 128)
    scores = sc_ref[...]
    ix = ix_ref[...]

    def vbody(t, lh):
        lo, hi = lh
        mid = 0.5 * (lo + hi)
        cnt = jnp.sum(jnp.where(scores >= mid, 1, 0))
        big = cnt >= PRE_NMS_TOPK
        return jnp.where(big, mid, lo), jnp.where(big, hi, mid)

    vstar, _ = lax.fori_loop(0, 64, vbody, (jnp.float32(-2.0), jnp.float32(2.0)))
    gt = scores > vstar
    ties = scores == vstar
    need = PRE_NMS_TOPK - jnp.sum(jnp.where(gt, 1, 0))

    def tbody(t, T):
        Tp = T + lax.shift_left(jnp.int32(1), 20 - t)
        cnt = jnp.sum(jnp.where(ties & (ix < Tp), 1, 0))
        return jnp.where(cnt <= need, Tp, T)

    tstar = lax.fori_loop(0, 21, tbody, jnp.int32(0))
    kept = gt | (ties & (ix < tstar))

    w = px2_ref[...] - px1_ref[...]
    h = py2_ref[...] - py1_ref[...]
    cx = px1_ref[...] + 0.5 * w
    cy = py1_ref[...] + 0.5 * h
    dw = jnp.minimum(dw_ref[...], _BBOX_CLIP)
    dh = jnp.minimum(dh_ref[...], _BBOX_CLIP)
    pcx = dx_ref[...] * w + cx
    pcy = dy_ref[...] * h + cy
    pw = jnp.exp(dw) * w
    ph = jnp.exp(dh) * h
    x1 = pcx - 0.5 * pw
    y1 = pcy - 0.5 * ph
    x2 = pcx + 0.5 * pw
    y2 = pcy + 0.5 * ph

    neg_inf = jnp.float32(-jnp.inf)
    mc = jnp.maximum(
        jnp.max(jnp.where(kept, x1, neg_inf)),
        jnp.maximum(jnp.max(jnp.where(kept, y1, neg_inf)),
                    jnp.maximum(jnp.max(jnp.where(kept, x2, neg_inf)),
                                jnp.max(jnp.where(kept, y2, neg_inf)))))
    max_coord = mc + 1.0
    labels = ((ix - (ix // C) * C) + 1).astype(jnp.float32)
    offs = labels * max_coord
    x1o = x1 + offs
    y1o = y1 + offs
    x2o = x2 + offs
    y2o = y2 + offs
    areas = (x2o - x1o) * (y2o - y1o)

    work0 = jnp.where(kept, scores, neg_inf)
    out0 = jnp.zeros((128, 8), jnp.float32)
    orow = lax.broadcasted_iota(jnp.int32, (128, 8), 0)
    ocol = lax.broadcasted_iota(jnp.int32, (128, 8), 1)
    big_i = jnp.int32(2**30)

    def body(j, carry):
        work, out_acc = carry
        m = jnp.max(work)
        iix = jnp.min(jnp.where(work == m, ix, big_i))
        sel = (work == m) & (ix == iix)
        xi1 = jnp.sum(jnp.where(sel, x1o, 0.0))
        yi1 = jnp.sum(jnp.where(sel, y1o, 0.0))
        xi2 = jnp.sum(jnp.where(sel, x2o, 0.0))
        yi2 = jnp.sum(jnp.where(sel, y2o, 0.0))
        ai = jnp.sum(jnp.where(sel, areas, 0.0))
        bx1 = jnp.sum(jnp.where(sel, x1, 0.0))
        by1 = jnp.sum(jnp.where(sel, y1, 0.0))
        bx2 = jnp.sum(jnp.where(sel, x2, 0.0))
        by2 = jnp.sum(jnp.where(sel, y2, 0.0))
        valid = m > 0.0
        row = (jnp.where(ocol == 0, jnp.where(valid, bx1, 0.0), 0.0)
               + jnp.where(ocol == 1, jnp.where(valid, by1, 0.0), 0.0)
               + jnp.where(ocol == 2, jnp.where(valid, bx2, 0.0), 0.0)
               + jnp.where(ocol == 3, jnp.where(valid, by2, 0.0), 0.0)
               + jnp.where(ocol == 4, jnp.where(valid, m, 0.0), 0.0))
        out_acc = jnp.where(orow == j, row, out_acc)
        xx1 = jnp.maximum(x1o, xi1)
        yy1 = jnp.maximum(y1o, yi1)
        xx2 = jnp.minimum(x2o, xi2)
        yy2 = jnp.minimum(y2o, yi2)
        inter = jnp.clip(xx2 - xx1, 0.0) * jnp.clip(yy2 - yy1, 0.0)
        iou = inter / (areas + ai - inter + 1e-9)
        work = jnp.where(iou > NMS_THRESH, neg_inf, work)
        return work, out_acc

    _, out_acc = lax.fori_loop(0, DET_PER_IM, body, (work0, out0))
    out_ref[...] = out_acc[:DET_PER_IM, :5]


def _nms(csc, cix, cd, cp):
    shp = (_CAP2 // 128, 128)
    args = [csc.reshape(shp), cix.reshape(shp)]
    args += [cd[:, k].reshape(shp) for k in range(4)]
    args += [cp[:, k].reshape(shp) for k in range(4)]
    return pl.pallas_call(
        _nms_kernel,
        out_shape=jax.ShapeDtypeStruct((DET_PER_IM, 5), jnp.float32),
    )(*args)


@jax.jit
def kernel(label_pre, bbox_pre, proposals):
    masked = _masked_scores(label_pre, bbox_pre, proposals)
    flat = masked.reshape(_TOT)

    hist = _hist_call(flat)
    counts = hist.reshape(_NW, _NB, 16).sum(axis=(0, 2))
    cum = jnp.cumsum(counts[::-1])[::-1]
    ks = jnp.arange(_NB, dtype=jnp.int32)
    feas = (cum >= PRE_NMS_TOPK) & (ks >= 1)
    kstar = jnp.max(jnp.where(feas, ks, 1)).astype(jnp.int32)

    csc, cix, cdf, cpf = _compact_call(
        flat,
        jnp.full((16,), kstar, jnp.int32),
        bbox_pre.reshape(N * (C + 1) * 4 // 128, 128),
        proposals.reshape(N * 4 // 128, 128),
        jnp.zeros((64,), jnp.float32),
    )
    return _nms(csc, cix, cdf.reshape(_CAP2, 4), cpf.reshape(_CAP2, 4))


# speedup vs baseline: 28.0286x; 1.1232x over previous
import functools

import jax
import jax.numpy as jnp
import numpy as np
from jax import lax
from jax.experimental import pallas as pl
from jax.experimental.pallas import tpu as pltpu
from jax.experimental.pallas import tpu_sc as plsc

N = 20000
C = 80
SCORE_THRESH = 0.01
NMS_THRESH = 0.5
DET_PER_IM = 100
PRE_NMS_TOPK = 1000
_BBOX_CLIP = float(np.log(1000.0 / 16.0))

_BLK = 2000
_NW = 32
_NC = 2
_TOT = N * C
_CHUNK = _TOT // _NW
_NB = 2048
_HSCALE = (_NB - 2) / 0.99
_CAPC = 1024
_CAP2 = _NC * _CAPC


def _scores_kernel(label_ref, bbox_ref, prop_ref, out_ref):
    lab = label_ref[...]
    m = jnp.max(lab, axis=1, keepdims=True)
    e = jnp.exp(lab - m)
    s = jnp.sum(e, axis=1, keepdims=True)
    scores = (e / s)[:, 1:]

    bb = bbox_ref[...]
    rr = lax.broadcasted_iota(jnp.int32, ((C + 1) * 4, C), 0)
    cc = lax.broadcasted_iota(jnp.int32, ((C + 1) * 4, C), 1)

    mask = jnp.uint32(0xFFFF0000)
    hi = lax.bitcast_convert_type(
        lax.bitcast_convert_type(bb, jnp.uint32) & mask, jnp.float32)
    r1 = bb - hi
    mi = lax.bitcast_convert_type(
        lax.bitcast_convert_type(r1, jnp.uint32) & mask, jnp.float32)
    r2 = r1 - mi
    chunks = [hi.astype(jnp.bfloat16), mi.astype(jnp.bfloat16),
              r2.astype(jnp.bfloat16)]

    def _sel(k):
        s = (rr == (cc + 1) * 4 + k).astype(jnp.bfloat16)
        out = jnp.zeros((_BLK, C), jnp.float32)
        for ch in chunks:
            out = out + lax.dot_general(ch, s, (((1,), (0,)), ((), ())),
                                        preferred_element_type=jnp.float32)
        return out

    dx = _sel(0)
    dy = _sel(1)
    dw = jnp.minimum(_sel(2), _BBOX_CLIP)
    dh = jnp.minimum(_sel(3), _BBOX_CLIP)

    p = prop_ref[...]
    w = p[:, 2:3] - p[:, 0:1]
    h = p[:, 3:4] - p[:, 1:2]
    cx = p[:, 0:1] + 0.5 * w
    cy = p[:, 1:2] + 0.5 * h

    pcx = dx * w + cx
    pcy = dy * h + cy
    pw = jnp.exp(dw) * w
    ph = jnp.exp(dh) * h
    x1 = pcx - 0.5 * pw
    y1 = pcy - 0.5 * ph
    x2 = pcx + 0.5 * pw
    y2 = pcy + 0.5 * ph
    area = (y2 - y1) * (x2 - x1)

    valid = (scores > SCORE_THRESH) & (area > 0.1)
    out_ref[...] = jnp.where(valid, scores, -1.0)


def _masked_scores(label_pre, bbox_pre, proposals):
    grid = N // _BLK
    return pl.pallas_call(
        _scores_kernel,
        grid=(grid,),
        in_specs=[
            pl.BlockSpec((_BLK, C + 1), lambda i: (i, 0)),
            pl.BlockSpec((_BLK, (C + 1) * 4), lambda i: (i, 0)),
            pl.BlockSpec((_BLK, 4), lambda i: (i, 0)),
        ],
        out_specs=pl.BlockSpec((_BLK, C), lambda i: (i, 0)),
        out_shape=jax.ShapeDtypeStruct((N, C), jnp.float32),
    )(label_pre, bbox_pre, proposals)


def _bucket_of(v):
    b = ((v - SCORE_THRESH) * _HSCALE).astype(jnp.int32) + 1
    return jnp.clip(b, 0, _NB - 1)


def _sc_mesh():
    return plsc.VectorSubcoreMesh(core_axis_name="c", subcore_axis_name="s",
                                  num_cores=_NC, num_subcores=_NW // _NC)


def _hist_body(sc_hbm, out_hbm, chunk_v, hist_v):
    cid = lax.axis_index("c")
    sid = lax.axis_index("s")
    wid = sid * _NC + cid
    pltpu.sync_copy(sc_hbm.at[pl.ds(pl.multiple_of(wid * _CHUNK, 8), _CHUNK)], chunk_v)

    zero = jnp.zeros((16,), jnp.int32)

    def zbody(i, carry):
        hist_v[pl.ds(i * 16, 16)] = zero
        return carry

    lax.fori_loop(0, _NB, zbody, 0)

    lanes = lax.iota(jnp.int32, 16)
    ones = jnp.ones((16,), jnp.int32)

    @plsc.parallel_loop(0, _CHUNK // 16, unroll=8)
    def _(i):
        v = chunk_v[pl.ds(i * 16, 16)]
        b = _bucket_of(v)
        plsc.addupdate_scatter(hist_v, [b * 16 + lanes], ones)
    pltpu.sync_copy(hist_v, out_hbm.at[wid])


@functools.cache
def _hist_sc():
    return pl.kernel(
        _hist_body,
        out_type=jax.ShapeDtypeStruct((_NW, _NB * 16), jnp.int32),
        mesh=_sc_mesh(),
        compiler_params=pltpu.CompilerParams(needs_layout_passes=False),
        scratch_types=[
            pltpu.VMEM((_CHUNK,), jnp.float32),
            pltpu.VMEM((_NB * 16,), jnp.int32),
        ],
    )


def _hist_call(flat):
    return _hist_sc()(flat)


def _compact_body(sc_hbm, kst_hbm, bb128_hbm, pp128_hbm, zflat_hbm,
                osc_hbm, oix_hbm, od_hbm, op_hbm,
                chunk_v, sbuf, ibuf, kst_v, zflat_v, rows_d, rows_p,
                grow_d, grow_p, cnt_smem, sem):
    cid = lax.axis_index("c")
    sid = lax.axis_index("s")
    wid = sid * _NC + cid

    neg1 = jnp.full((16,), -1.0, jnp.float32)
    izero = jnp.zeros((16,), jnp.int32)

    for t in range(4):
        sbuf[pl.ds(t * 16, 16)] = neg1
        ibuf[pl.ds(t * 16, 16)] = izero
    zoff = pl.multiple_of(cid * _CAPC + sid * (_CAPC // 16), 8)
    pltpu.sync_copy(sbuf.at[pl.ds(0, _CAPC // 16)], osc_hbm.at[pl.ds(zoff, _CAPC // 16)])
    pltpu.sync_copy(ibuf.at[pl.ds(0, _CAPC // 16)], oix_hbm.at[pl.ds(zoff, _CAPC // 16)])
    pltpu.sync_copy(zflat_hbm, zflat_v)
    for t in range(4):
        zf = pl.multiple_of(zoff * 4 + t * 64, 8)
        pltpu.sync_copy(zflat_v, od_hbm.at[pl.ds(zf, 64)])
        pltpu.sync_copy(zflat_v, op_hbm.at[pl.ds(zf, 64)])

    @pl.when(sid == 0)
    def _():
        cnt_smem[0] = 0

    pltpu.sync_copy(kst_hbm, kst_v)
    base_elem = wid * _CHUNK
    pltpu.sync_copy(sc_hbm.at[pl.ds(pl.multiple_of(base_elem, 8), _CHUNK)], chunk_v)
    plsc.subcore_barrier()

    kvec = kst_v[...]
    lanes = lax.iota(jnp.int32, 16)

    capv = jnp.full((16,), _CAPC, jnp.int32)

    @plsc.parallel_loop(0, _CHUNK // 16, unroll=4,
                        carry=jnp.zeros((16,), jnp.int32))
    def wvec(i, wv):
        v = chunk_v[pl.ds(i * 16, 16)]
        m = _bucket_of(v) >= kvec
        pc = plsc.cumsum(jnp.where(m, 1, 0))
        pos = wv + pc - 1
        plsc.store_scatter(sbuf, [pos], v, mask=m)
        plsc.store_scatter(ibuf, [pos], base_elem + i * 16 + lanes, mask=m)
        return jnp.minimum(wv + plsc.all_reduce_population_count(m), capv)

    wcnt = jnp.max(wvec)

    plsc.store_scatter(sbuf, [wcnt + lanes], neg1)
    plsc.store_scatter(ibuf, [wcnt + lanes], izero)
    wpad = ((wcnt + 15) // 16) * 16
    mybase = plsc.fetch_and_add(cnt_smem.at[0], wpad, subcore_id=0)

    lane4 = lax.iota(jnp.int32, 16)

    def wbody(j, carry):
        off = mybase + j * 16

        @pl.when(off <= _CAPC - 16)
        def _():
            dst = pl.multiple_of(cid * _CAPC + off, 8)
            pltpu.sync_copy(sbuf.at[pl.ds(j * 16, 16)], osc_hbm.at[pl.ds(dst, 16)])
            pltpu.sync_copy(ibuf.at[pl.ds(j * 16, 16)], oix_hbm.at[pl.ds(dst, 16)])
            ivec = ibuf[pl.ds(j * 16, 16)]
            n = ivec // C
            cls = ivec - n * C
            offd = n * ((C + 1) * 4) + (cls + 1) * 4
            offp = n * 4
            pltpu.async_copy(bb128_hbm.at[lax.shift_right_logical(offd, 7)],
                             grow_d, sem).wait()
            pltpu.async_copy(pp128_hbm.at[lax.shift_right_logical(offp, 7)],
                             grow_p, sem).wait()
            cold = offd & 127
            colp = offp & 127
            for k in range(4):
                vd = plsc.load_gather(grow_d, [lane4, cold + k])
                vp = plsc.load_gather(grow_p, [lane4, colp + k])
                plsc.store_scatter(rows_d, [lane4 * 4 + k], vd)
                plsc.store_scatter(rows_p, [lane4 * 4 + k], vp)
            pltpu.sync_copy(rows_d, od_hbm.at[pl.ds(pl.multiple_of(dst * 4, 8), 64)])
            pltpu.sync_copy(rows_p, op_hbm.at[pl.ds(pl.multiple_of(dst * 4, 8), 64)])

        return carry

    lax.fori_loop(0, wpad // 16, wbody, 0)


@functools.cache
def _compact_sc():
    return pl.kernel(
        _compact_body,
        out_type=[
            jax.ShapeDtypeStruct((_CAP2,), jnp.float32),
            jax.ShapeDtypeStruct((_CAP2,), jnp.int32),
            jax.ShapeDtypeStruct((_CAP2 * 4,), jnp.float32),
            jax.ShapeDtypeStruct((_CAP2 * 4,), jnp.float32),
        ],
        mesh=_sc_mesh(),
        compiler_params=pltpu.CompilerParams(needs_layout_passes=False),
        scratch_types=[
            pltpu.VMEM((_CHUNK,), jnp.float32),
            pltpu.VMEM((_CAPC + 16,), jnp.float32),
            pltpu.VMEM((_CAPC + 16,), jnp.int32),
            pltpu.VMEM((16,), jnp.int32),
            pltpu.VMEM((64,), jnp.float32),
            pltpu.VMEM((64,), jnp.float32),
            pltpu.VMEM((64,), jnp.float32),
            pltpu.VMEM((16, 128), jnp.float32),
            pltpu.VMEM((16, 128), jnp.float32),
            pltpu.SMEM((1,), jnp.int32),
            pltpu.SemaphoreType.DMA,
        ],
    )


def _compact_call(flat, kst, bb128, pp128, zflat):
    return _compact_sc()(flat, kst, bb128, pp128, zflat)


def _nms_kernel(sc_ref, ix_ref, dx_ref, dy_ref, dw_ref, dh_ref,
                px1_ref, py1_ref, px2_ref, py2_ref, out_ref):
    shape = (_CAP2 // 128, 128)
    scores = sc_ref[...]
    ix = ix_ref[...]

    def vbody(t, lh):
        lo, hi = lh
        mid = 0.5 * (lo + hi)
        cnt = jnp.sum(jnp.where(scores >= mid, 1, 0))
        big = cnt >= PRE_NMS_TOPK
        return jnp.where(big, mid, lo), jnp.where(big, hi, mid)

    vstar, _ = lax.fori_loop(0, 64, vbody, (jnp.float32(-2.0), jnp.float32(2.0)))
    gt = scores > vstar
    ties = scores == vstar
    need = PRE_NMS_TOPK - jnp.sum(jnp.where(gt, 1, 0))

    def tbody(t, T):
        Tp = T + lax.shift_left(jnp.int32(1), 20 - t)
        cnt = jnp.sum(jnp.where(ties & (ix < Tp), 1, 0))
        return jnp.where(cnt <= need, Tp, T)

    tstar = lax.fori_loop(0, 21, tbody, jnp.int32(0))
    kept = gt | (ties & (ix < tstar))

    w = px2_ref[...] - px1_ref[...]
    h = py2_ref[...] - py1_ref[...]
    cx = px1_ref[...] + 0.5 * w
    cy = py1_ref[...] + 0.5 * h
    dw = jnp.minimum(dw_ref[...], _BBOX_CLIP)
    dh = jnp.minimum(dh_ref[...], _BBOX_CLIP)
    pcx = dx_ref[...] * w + cx
    pcy = dy_ref[...] * h + cy
    pw = jnp.exp(dw) * w
    ph = jnp.exp(dh) * h
    x1 = pcx - 0.5 * pw
    y1 = pcy - 0.5 * ph
    x2 = pcx + 0.5 * pw
    y2 = pcy + 0.5 * ph

    neg_inf = jnp.float32(-jnp.inf)
    mc = jnp.maximum(
        jnp.max(jnp.where(kept, x1, neg_inf)),
        jnp.maximum(jnp.max(jnp.where(kept, y1, neg_inf)),
                    jnp.maximum(jnp.max(jnp.where(kept, x2, neg_inf)),
                                jnp.max(jnp.where(kept, y2, neg_inf)))))
    max_coord = mc + 1.0
    labels = ((ix - (ix // C) * C) + 1).astype(jnp.float32)
    offs = labels * max_coord
    x1o = x1 + offs
    y1o = y1 + offs
    x2o = x2 + offs
    y2o = y2 + offs
    areas = (x2o - x1o) * (y2o - y1o)

    work0 = jnp.where(kept, scores, neg_inf)
    out0 = jnp.zeros((128, 8), jnp.float32)
    orow = lax.broadcasted_iota(jnp.int32, (128, 8), 0)
    ocol = lax.broadcasted_iota(jnp.int32, (128, 8), 1)
    big_i = jnp.int32(2**30)

    def body(j, carry):
        work, out_acc = carry
        m = jnp.max(work)
        iix = jnp.min(jnp.where(work == m, ix, big_i))
        sel = (work == m) & (ix == iix)
        xi1 = jnp.sum(jnp.where(sel, x1o, 0.0))
        yi1 = jnp.sum(jnp.where(sel, y1o, 0.0))
        xi2 = jnp.sum(jnp.where(sel, x2o, 0.0))
        yi2 = jnp.sum(jnp.where(sel, y2o, 0.0))
        ai = jnp.sum(jnp.where(sel, areas, 0.0))
        bx1 = jnp.sum(jnp.where(sel, x1, 0.0))
        by1 = jnp.sum(jnp.where(sel, y1, 0.0))
        bx2 = jnp.sum(jnp.where(sel, x2, 0.0))
        by2 = jnp.sum(jnp.where(sel, y2, 0.0))
        valid = m > 0.0
        row = (jnp.where(ocol == 0, jnp.where(valid, bx1, 0.0), 0.0)
               + jnp.where(ocol == 1, jnp.where(valid, by1, 0.0), 0.0)
               + jnp.where(ocol == 2, jnp.where(valid, bx2, 0.0), 0.0)
               + jnp.where(ocol == 3, jnp.where(valid, by2, 0.0), 0.0)
               + jnp.where(ocol == 4, jnp.where(valid, m, 0.0), 0.0))
        out_acc = jnp.where(orow == j, row, out_acc)
        xx1 = jnp.maximum(x1o, xi1)
        yy1 = jnp.maximum(y1o, yi1)
        xx2 = jnp.minimum(x2o, xi2)
        yy2 = jnp.minimum(y2o, yi2)
        inter = jnp.clip(xx2 - xx1, 0.0) * jnp.clip(yy2 - yy1, 0.0)
        iou = inter / (areas + ai - inter + 1e-9)
        work = jnp.where(iou > NMS_THRESH, neg_inf, work)
        return work, out_acc

    _, out_acc = lax.fori_loop(0, DET_PER_IM, body, (work0, out0))
    out_ref[...] = out_acc[:DET_PER_IM, :5]


def _nms(csc, cix, cd, cp):
    shp = (_CAP2 // 128, 128)
    args = [csc.reshape(shp), cix.reshape(shp)]
    args += [cd[:, k].reshape(shp) for k in range(4)]
    args += [cp[:, k].reshape(shp) for k in range(4)]
    return pl.pallas_call(
        _nms_kernel,
        out_shape=jax.ShapeDtypeStruct((DET_PER_IM, 5), jnp.float32),
    )(*args)


@jax.jit
def kernel(label_pre, bbox_pre, proposals):
    masked = _masked_scores(label_pre, bbox_pre, proposals)
    flat = masked.reshape(_TOT)

    hist = _hist_call(flat)
    counts = hist.reshape(_NW, _NB, 16).sum(axis=(0, 2))
    cum = jnp.cumsum(counts[::-1])[::-1]
    ks = jnp.arange(_NB, dtype=jnp.int32)
    feas = (cum >= PRE_NMS_TOPK) & (ks >= 1)
    kstar = jnp.max(jnp.where(feas, ks, 1)).astype(jnp.int32)

    csc, cix, cdf, cpf = _compact_call(
        flat,
        jnp.full((16,), kstar, jnp.int32),
        bbox_pre.reshape(N * (C + 1) * 4 // 128, 128),
        proposals.reshape(N * 4 // 128, 128),
        jnp.zeros((64,), jnp.float32),
    )
    return _nms(csc, cix, cdf.reshape(_CAP2, 4), cpf.reshape(_CAP2, 4))
